# Initial kernel scaffold; baseline (speedup 1.0000x reference)
import functools

import jax
import jax.numpy as jnp
import numpy as np
from jax import lax
from jax.experimental import pallas as pl
from jax.experimental.pallas import tpu as pltpu
from jax.experimental.pallas import tpu_sc as plsc

N_NODES = 10000
E_EDGES = 320000
NODE_DIM = 128
EDGE_DIM = 16
HIDDEN = 128
N_HEADS = 4
HEAD_DIM = 32
INV_SQRT_HD = 1.0 / float(np.sqrt(HEAD_DIM))

NW = 32
EPW = E_EDGES // NW
C = 80
NCH = EPW // C
ACC_W = 144
ZR = 125
RPT = N_NODES // 16

NBLK = 1000
EBLK = 2000

_mesh = plsc.VectorSubcoreMesh(core_axis_name="c", subcore_axis_name="s")


def _proj_body(x_ref, w_ref, b_ref, q_ref, k_ref, v_ref, p_ref):
    r = jnp.dot(x_ref[...], w_ref[...], preferred_element_type=jnp.float32)
    r = r + b_ref[...]
    q_ref[...] = r[:, 0:128]
    k_ref[...] = r[:, 128:256]
    v_ref[...] = r[:, 256:384]
    p_ref[...] = r[:, 384:512]


def _ep_body(ef_ref, we_ref, be_ref, ep_ref):
    ep_ref[...] = (
        jnp.dot(ef_ref[...], we_ref[...], preferred_element_type=jnp.float32)
        + be_ref[...]
    )


def _node_body(p0_ref, p1_ref, x_ref, pp_ref, exp_ref, wga_ref, wgb_ref,
               bg_ref, wo_ref, bo_ref, wab_ref, g1_ref, b1_ref,
               no_ref, a_ref, b_ref):
    num = p0_ref[:, 0:128] + p1_ref[:, 0:128]
    den = p0_ref[:, 128:132] + p1_ref[:, 128:132]
    den_b = jnp.dot(den, exp_ref[...], preferred_element_type=jnp.float32)
    agg = num / (den_b + 1e-10)
    x = x_ref[...]
    gate = jax.nn.sigmoid(
        jnp.dot(agg, wga_ref[...], preferred_element_type=jnp.float32)
        + jnp.dot(pp_ref[...], wgb_ref[...], preferred_element_type=jnp.float32)
        + bg_ref[...]
    )
    upd = jnp.dot(agg, wo_ref[...], preferred_element_type=jnp.float32) + bo_ref[...]
    y = x + gate * upd + (1.0 - gate) * x
    mu = jnp.mean(y, axis=1, keepdims=True)
    yc = y - mu
    var = jnp.mean(yc * yc, axis=1, keepdims=True)
    no = yc * lax.rsqrt(var + 1e-5) * g1_ref[...] + b1_ref[...]
    no_ref[...] = no
    ab = jnp.dot(no, wab_ref[...], preferred_element_type=jnp.float32)
    a_ref[...] = ab[:, 0:128]
    b_ref[...] = ab[:, 128:256]


def _edge_body(pre_ref, ef_ref, w1c_ref, b1u_ref, w2_ref, b2u_ref,
               g2_ref, b2n_ref, out_ref):
    ef = ef_ref[...]
    h = jnp.maximum(
        pre_ref[...]
        + jnp.dot(ef, w1c_ref[...], preferred_element_type=jnp.float32)
        + b1u_ref[...],
        0.0,
    )
    eu = jnp.dot(h, w2_ref[...], preferred_element_type=jnp.float32) + b2u_ref[...]
    y = ef + eu
    mu = jnp.mean(y, axis=1, keepdims=True)
    yc = y - mu
    var = jnp.mean(yc * yc, axis=1, keepdims=True)
    out_ref[...] = yc * lax.rsqrt(var + 1e-5) * g2_ref[...] + b2n_ref[...]


@functools.partial(
    pl.kernel,
    out_type=jax.ShapeDtypeStruct((2, N_NODES, ACC_W), jnp.float32),
    mesh=_mesh,
    scratch_types=[
        pltpu.VMEM((C,), jnp.int32),
        pltpu.VMEM((C,), jnp.int32),
        pltpu.VMEM((C, 128), jnp.float32),
        pltpu.VMEM((C, 128), jnp.float32),
        pltpu.VMEM((C, 128), jnp.float32),
        pltpu.VMEM((C, 128), jnp.float32),
        pltpu.VMEM((C, ACC_W), jnp.float32),
        pltpu.VMEM((ZR, ACC_W), jnp.float32),
        pltpu.VMEM_SHARED((N_NODES, ACC_W), jnp.float32),
        pltpu.SemaphoreType.DMA,
        pltpu.SemaphoreType.DMA,
        pltpu.SemaphoreType.DMA,
        pltpu.SemaphoreType.DMA,
    ],
)
def _attn_sc(q_hbm, k_hbm, v_hbm, ep_hbm, src_hbm, dst_hbm, out_hbm,
             src_v, dst_v, qv, kv, ev, vv, wv, zv, acc, s0, s1, s2, s3):
    c = lax.axis_index("c")
    s = lax.axis_index("s")
    wid = c * 16 + s

    def zbody(i, carry):
        for j in range(ACC_W // 16):
            zv[i, pl.ds(j * 16, 16)] = jnp.zeros((16,), jnp.float32)
        return carry

    lax.fori_loop(0, ZR, zbody, 0)
    for rb in range(RPT // ZR):
        pltpu.sync_copy(zv, acc.at[pl.ds(s * RPT + rb * ZR, ZR)])
    plsc.subcore_barrier()

    lane = lax.broadcasted_iota(jnp.int32, (16,), 0)

    def chunk(t, carry):
        base = pl.multiple_of(wid * EPW + t * C, 8)
        pltpu.sync_copy(src_hbm.at[pl.ds(base, C)], src_v)
        pltpu.sync_copy(dst_hbm.at[pl.ds(base, C)], dst_v)
        cpe = pltpu.async_copy(ep_hbm.at[pl.ds(base, C), :], ev, s3)
        cp0 = pltpu.async_copy(q_hbm.at[dst_v], qv, s0)
        cp1 = pltpu.async_copy(k_hbm.at[src_v], kv, s1)
        cp2 = pltpu.async_copy(v_hbm.at[src_v], vv, s2)
        cpe.wait()
        cp0.wait()
        cp1.wait()
        cp2.wait()

        def ebody(i, ecarry):
            svec = jnp.zeros((16,), jnp.float32)
            for h in range(4):
                o0 = h * 32
                o1 = h * 32 + 16
                t0 = qv[i, pl.ds(o0, 16)] * (kv[i, pl.ds(o0, 16)] + ev[i, pl.ds(o0, 16)])
                t1 = qv[i, pl.ds(o1, 16)] * (kv[i, pl.ds(o1, 16)] + ev[i, pl.ds(o1, 16)])
                th = jnp.sum(t0 + t1) * INV_SQRT_HD
                sh = jnp.exp(jnp.full((16,), th, jnp.float32))
                wv[i, pl.ds(o0, 16)] = sh * vv[i, pl.ds(o0, 16)]
                wv[i, pl.ds(o1, 16)] = sh * vv[i, pl.ds(o1, 16)]
                svec = jnp.where(lane == h, sh, svec)
            wv[i, pl.ds(128, 16)] = svec
            return ecarry

        lax.fori_loop(0, C, ebody, 0)
        pltpu.sync_copy(wv, acc.at[dst_v], add=True)
        return carry

    lax.fori_loop(0, NCH, chunk, 0)
    plsc.subcore_barrier()
    for rb in range(RPT // ZR):
        r0 = s * RPT + rb * ZR
        pltpu.sync_copy(acc.at[pl.ds(r0, ZR)], out_hbm.at[c, pl.ds(r0, ZR), :])


@functools.partial(
    pl.kernel,
    out_type=jax.ShapeDtypeStruct((E_EDGES, 128), jnp.float32),
    mesh=_mesh,
    scratch_types=[
        pltpu.VMEM((C,), jnp.int32),
        pltpu.VMEM((C,), jnp.int32),
        pltpu.VMEM((C, 128), jnp.float32),
        pltpu.VMEM((C, 128), jnp.float32),
        pltpu.VMEM((C, 128), jnp.float32),
        pltpu.SemaphoreType.DMA,
        pltpu.SemaphoreType.DMA,
    ],
)
def _gather_sc(a_hbm, b_hbm, src_hbm, dst_hbm, out_hbm,
               src_v, dst_v, av, bv, pv, s0, s1):
    wid = lax.axis_index("c") * 16 + lax.axis_index("s")

    def chunk(t, carry):
        base = pl.multiple_of(wid * EPW + t * C, 8)
        pltpu.sync_copy(src_hbm.at[pl.ds(base, C)], src_v)
        pltpu.sync_copy(dst_hbm.at[pl.ds(base, C)], dst_v)
        cp0 = pltpu.async_copy(a_hbm.at[src_v], av, s0)
        cp1 = pltpu.async_copy(b_hbm.at[dst_v], bv, s1)
        cp0.wait()
        cp1.wait()

        def ebody(i, ecarry):
            for j in range(8):
                pv[i, pl.ds(j * 16, 16)] = (
                    av[i, pl.ds(j * 16, 16)] + bv[i, pl.ds(j * 16, 16)]
                )
            return ecarry

        lax.fori_loop(0, C, ebody, 0)
        pltpu.sync_copy(pv, out_hbm.at[pl.ds(base, C), :])
        return carry

    lax.fori_loop(0, NCH, chunk, 0)


def _full(shape):
    return pl.BlockSpec(shape, lambda i: (0,) * len(shape))


def kernel(node_feats, edge_feats, edge_index, Wq, bq, Wk, bk, Wv, bv, We, be,
           Wg, bg, Wo, bo, Wu1, bu1, Wu2, bu2, g1, b1, g2, b2):
    N, E = N_NODES, E_EDGES
    src = edge_index[0]
    dst = edge_index[1]

    w_qkvp = jnp.concatenate([Wq, Wk, Wv, Wo], axis=1)
    b_qkvp = jnp.concatenate([bq, bk, bv, bo]).reshape(1, 512)

    Q, K, V, P = pl.pallas_call(
        _proj_body,
        grid=(N // NBLK,),
        in_specs=[
            pl.BlockSpec((NBLK, 128), lambda i: (i, 0)),
            _full((128, 512)),
            _full((1, 512)),
        ],
        out_specs=[pl.BlockSpec((NBLK, 128), lambda i: (i, 0))] * 4,
        out_shape=[jax.ShapeDtypeStruct((N, 128), jnp.float32)] * 4,
    )(node_feats, w_qkvp, b_qkvp)

    Ep = pl.pallas_call(
        _ep_body,
        grid=(E // EBLK,),
        in_specs=[
            pl.BlockSpec((EBLK, 16), lambda i: (i, 0)),
            _full((16, 128)),
            _full((1, 128)),
        ],
        out_specs=pl.BlockSpec((EBLK, 128), lambda i: (i, 0)),
        out_shape=jax.ShapeDtypeStruct((E, 128), jnp.float32),
    )(edge_feats, We, be.reshape(1, 128))

    parts = _attn_sc(Q, K, V, Ep, src, dst)

    expander = jnp.kron(jnp.eye(4, dtype=jnp.float32),
                        jnp.ones((1, 32), jnp.float32))
    node_out, A, B = pl.pallas_call(
        _node_body,
        grid=(N // NBLK,),
        in_specs=[
            pl.BlockSpec((NBLK, ACC_W), lambda i: (i, 0)),
            pl.BlockSpec((NBLK, ACC_W), lambda i: (i, 0)),
            pl.BlockSpec((NBLK, 128), lambda i: (i, 0)),
            pl.BlockSpec((NBLK, 128), lambda i: (i, 0)),
            _full((4, 128)),
            _full((128, 128)),
            _full((128, 128)),
            _full((1, 128)),
            _full((128, 128)),
            _full((1, 128)),
            _full((128, 256)),
            _full((1, 128)),
            _full((1, 128)),
        ],
        out_specs=[pl.BlockSpec((NBLK, 128), lambda i: (i, 0))] * 3,
        out_shape=[jax.ShapeDtypeStruct((N, 128), jnp.float32)] * 3,
    )(parts[0], parts[1], node_feats, P, expander, Wg[:128], Wg[128:],
      bg.reshape(1, 128), Wo, bo.reshape(1, 128), Wu1[:256], g1.reshape(1, 128),
      b1.reshape(1, 128))

    prelim = _gather_sc(A, B, src, dst)

    edge_out = pl.pallas_call(
        _edge_body,
        grid=(E // EBLK,),
        in_specs=[
            pl.BlockSpec((EBLK, 128), lambda i: (i, 0)),
            pl.BlockSpec((EBLK, 16), lambda i: (i, 0)),
            _full((16, 128)),
            _full((1, 128)),
            _full((128, 16)),
            _full((1, 16)),
            _full((1, 16)),
            _full((1, 16)),
        ],
        out_specs=pl.BlockSpec((EBLK, 16), lambda i: (i, 0)),
        out_shape=jax.ShapeDtypeStruct((E, 16), jnp.float32),
    )(prelim, edge_feats, Wu1[256:], bu1.reshape(1, 128), Wu2,
      bu2.reshape(1, 16), g2.reshape(1, 16), b2.reshape(1, 16))

    return node_out, edge_out


# speedup vs baseline: 8.4464x; 8.4464x over previous
import functools

import jax
import jax.numpy as jnp
import numpy as np
from jax import lax
from jax.experimental import pallas as pl
from jax.experimental.pallas import tpu as pltpu
from jax.experimental.pallas import tpu_sc as plsc

N_NODES = 10000
E_EDGES = 320000
INV_SQRT_HD = 1.0 / float(np.sqrt(32))

NW = 32
NPAD = 10240
EPAD = 322560
EPW = EPAD // NW
CA = 48
NCHA = EPW // CA
CB = 96
NCHB = EPW // CB
ACC_W = 144
RPT = NPAD // 16

NBLK = 1024
EBLK = 2240

_mesh = plsc.VectorSubcoreMesh(core_axis_name="c", subcore_axis_name="s")


def _proj_body(x_ref, w_ref, b_ref, q_ref, k_ref, v_ref, p_ref):
    r = jnp.dot(x_ref[...], w_ref[...], preferred_element_type=jnp.float32)
    r = r + b_ref[...]
    q_ref[...] = r[:, 0:128]
    k_ref[...] = r[:, 128:256]
    v_ref[...] = r[:, 256:384]
    p_ref[...] = r[:, 384:512]


def _ep_body(ef_ref, we_ref, be_ref, ep_ref):
    ep_ref[...] = (
        jnp.dot(ef_ref[...], we_ref[...], preferred_element_type=jnp.float32)
        + be_ref[...]
    )


def _node_body(p0_ref, p1_ref, x_ref, pp_ref, exp_ref, wga_ref, wgb_ref,
               bg_ref, wo_ref, bo_ref, wab_ref, g1_ref, b1_ref,
               no_ref, a_ref, b_ref):
    num = p0_ref[:, 0:128] + p1_ref[:, 0:128]
    den = p0_ref[:, 128:132] + p1_ref[:, 128:132]
    den_b = jnp.dot(den, exp_ref[...], preferred_element_type=jnp.float32)
    agg = num / (den_b + 1e-10)
    x = x_ref[...]
    gate = jax.nn.sigmoid(
        jnp.dot(agg, wga_ref[...], preferred_element_type=jnp.float32)
        + jnp.dot(pp_ref[...], wgb_ref[...], preferred_element_type=jnp.float32)
        + bg_ref[...]
    )
    upd = jnp.dot(agg, wo_ref[...], preferred_element_type=jnp.float32) + bo_ref[...]
    y = x + gate * upd + (1.0 - gate) * x
    mu = jnp.mean(y, axis=1, keepdims=True)
    yc = y - mu
    var = jnp.mean(yc * yc, axis=1, keepdims=True)
    no = yc * lax.rsqrt(var + 1e-5) * g1_ref[...] + b1_ref[...]
    no_ref[...] = no
    ab = jnp.dot(no, wab_ref[...], preferred_element_type=jnp.float32)
    a_ref[...] = ab[:, 0:128]
    b_ref[...] = ab[:, 128:256]


def _edge_body(pre_ref, ef_ref, w1c_ref, b1u_ref, w2_ref, b2u_ref,
               g2_ref, b2n_ref, out_ref):
    ef = ef_ref[...]
    h = jnp.maximum(
        pre_ref[...]
        + jnp.dot(ef, w1c_ref[...], preferred_element_type=jnp.float32)
        + b1u_ref[...],
        0.0,
    )
    eu = jnp.dot(h, w2_ref[...], preferred_element_type=jnp.float32) + b2u_ref[...]
    y = ef + eu
    mu = jnp.mean(y, axis=1, keepdims=True)
    yc = y - mu
    var = jnp.mean(yc * yc, axis=1, keepdims=True)
    out_ref[...] = yc * lax.rsqrt(var + 1e-5) * g2_ref[...] + b2n_ref[...]


@functools.partial(
    pl.kernel,
    out_type=jax.ShapeDtypeStruct((2, NPAD, ACC_W), jnp.float32),
    mesh=_mesh,
    compiler_params=pltpu.CompilerParams(
        use_tc_tiling_on_sc=False, needs_layout_passes=False),
    scratch_types=[
        pltpu.VMEM((CA,), jnp.int32),
        pltpu.VMEM((CA,), jnp.int32),
        pltpu.VMEM((CA, 128), jnp.float32),
        pltpu.VMEM((CA, 128), jnp.float32),
        pltpu.VMEM((CA, 128), jnp.float32),
        pltpu.VMEM((CA, 128), jnp.float32),
        pltpu.VMEM((CA, ACC_W), jnp.float32),
        pltpu.VMEM_SHARED((NPAD, ACC_W), jnp.float32),
        pltpu.SemaphoreType.DMA,
        pltpu.SemaphoreType.DMA,
        pltpu.SemaphoreType.DMA,
        pltpu.SemaphoreType.DMA,
    ],
)
def _attn_sc(q_hbm, k_hbm, v_hbm, ep_hbm, src_hbm, dst_hbm, out_hbm,
             src_v, dst_v, qv, kv, ev, vv, wv, acc, s0, s1, s2, s3):
    c = lax.axis_index("c")
    s = lax.axis_index("s")
    wid = c * 16 + s
    zero16 = jnp.zeros((16,), jnp.float32)

    def zbody(i, carry):
        for j in range(ACC_W // 16):
            wv[i, pl.ds(j * 16, 16)] = zero16
        return carry

    lax.fori_loop(0, CA, zbody, 0)
    for rb in range(RPT // 40):
        pltpu.sync_copy(wv.at[pl.ds(0, 40)], acc.at[pl.ds(s * RPT + rb * 40, 40)])
    plsc.subcore_barrier()

    lane = lax.broadcasted_iota(jnp.int32, (16,), 0)

    def chunk(t, carry):
        base = pl.multiple_of(wid * EPW + t * CA, 8)
        pltpu.sync_copy(src_hbm.at[pl.ds(base, CA)], src_v)
        pltpu.sync_copy(dst_hbm.at[pl.ds(base, CA)], dst_v)
        cpe = pltpu.async_copy(ep_hbm.at[pl.ds(base, CA), :], ev, s3)
        cp0 = pltpu.async_copy(q_hbm.at[dst_v], qv, s0)
        cp1 = pltpu.async_copy(k_hbm.at[src_v], kv, s1)
        cp2 = pltpu.async_copy(v_hbm.at[src_v], vv, s2)
        cpe.wait()
        cp0.wait()
        cp1.wait()
        cp2.wait()

        for g in range(CA // 16):
            rows = lane + (g * 16)
            for h in range(4):
                a = zero16
                for d in range(h * 32, h * 32 + 32):
                    dsp = jnp.full((16,), d, jnp.int32)
                    qc = plsc.load_gather(qv, [rows, dsp])
                    kc = plsc.load_gather(kv, [rows, dsp])
                    ec = plsc.load_gather(ev, [rows, dsp])
                    a = a + qc * (kc + ec)
                sh = jnp.exp(a * INV_SQRT_HD)
                plsc.store_scatter(
                    wv, [rows, jnp.full((16,), 128 + h, jnp.int32)], sh)
                for d in range(h * 32, h * 32 + 32):
                    dsp = jnp.full((16,), d, jnp.int32)
                    vc = plsc.load_gather(vv, [rows, dsp])
                    plsc.store_scatter(wv, [rows, dsp], sh * vc)

        pltpu.sync_copy(wv, acc.at[dst_v], add=True)
        return carry

    lax.fori_loop(0, NCHA, chunk, 0)
    plsc.subcore_barrier()
    r0 = s * RPT
    pltpu.sync_copy(acc.at[pl.ds(r0, RPT)], out_hbm.at[c, pl.ds(r0, RPT), :])


@functools.partial(
    pl.kernel,
    out_type=jax.ShapeDtypeStruct((EPAD, 128), jnp.float32),
    mesh=_mesh,
    compiler_params=pltpu.CompilerParams(use_tc_tiling_on_sc=False),
    scratch_types=[
        pltpu.VMEM((CB,), jnp.int32),
        pltpu.VMEM((CB,), jnp.int32),
        pltpu.VMEM((CB, 128), jnp.float32),
        pltpu.VMEM((CB, 128), jnp.float32),
        pltpu.VMEM((CB, 128), jnp.float32),
        pltpu.SemaphoreType.DMA,
        pltpu.SemaphoreType.DMA,
    ],
)
def _gather_sc(a_hbm, b_hbm, src_hbm, dst_hbm, out_hbm,
               src_v, dst_v, av, bv, pv, s0, s1):
    wid = lax.axis_index("c") * 16 + lax.axis_index("s")

    def chunk(t, carry):
        base = pl.multiple_of(wid * EPW + t * CB, 8)
        pltpu.sync_copy(src_hbm.at[pl.ds(base, CB)], src_v)
        pltpu.sync_copy(dst_hbm.at[pl.ds(base, CB)], dst_v)
        cp0 = pltpu.async_copy(a_hbm.at[src_v], av, s0)
        cp1 = pltpu.async_copy(b_hbm.at[dst_v], bv, s1)
        cp0.wait()
        cp1.wait()

        def ebody(i, ecarry):
            for j in range(8):
                pv[i, pl.ds(j * 16, 16)] = (
                    av[i, pl.ds(j * 16, 16)] + bv[i, pl.ds(j * 16, 16)]
                )
            return ecarry

        lax.fori_loop(0, CB, ebody, 0)
        pltpu.sync_copy(pv, out_hbm.at[pl.ds(base, CB), :])
        return carry

    lax.fori_loop(0, NCHB, chunk, 0)


def _full(shape):
    return pl.BlockSpec(shape, lambda i: (0,) * len(shape))


def kernel(node_feats, edge_feats, edge_index, Wq, bq, Wk, bk, Wv, bv, We, be,
           Wg, bg, Wo, bo, Wu1, bu1, Wu2, bu2, g1, b1, g2, b2):
    N, E = N_NODES, E_EDGES
    src = jnp.pad(edge_index[0], (0, EPAD - E))
    dst = jnp.pad(edge_index[1], (0, EPAD - E), constant_values=NPAD - 1)
    nf_pad = jnp.pad(node_feats, ((0, NPAD - N), (0, 0)))
    ef_pad = jnp.pad(edge_feats, ((0, EPAD - E), (0, 0)))

    w_qkvp = jnp.concatenate([Wq, Wk, Wv, Wo], axis=1)
    b_qkvp = jnp.concatenate([bq, bk, bv, bo]).reshape(1, 512)

    Q, K, V, P = pl.pallas_call(
        _proj_body,
        grid=(NPAD // NBLK,),
        in_specs=[
            pl.BlockSpec((NBLK, 128), lambda i: (i, 0)),
            _full((128, 512)),
            _full((1, 512)),
        ],
        out_specs=[pl.BlockSpec((NBLK, 128), lambda i: (i, 0))] * 4,
        out_shape=[jax.ShapeDtypeStruct((NPAD, 128), jnp.float32)] * 4,
    )(nf_pad, w_qkvp, b_qkvp)

    Ep = pl.pallas_call(
        _ep_body,
        grid=(EPAD // EBLK,),
        in_specs=[
            pl.BlockSpec((EBLK, 16), lambda i: (i, 0)),
            _full((16, 128)),
            _full((1, 128)),
        ],
        out_specs=pl.BlockSpec((EBLK, 128), lambda i: (i, 0)),
        out_shape=jax.ShapeDtypeStruct((EPAD, 128), jnp.float32),
    )(ef_pad, We, be.reshape(1, 128))

    parts = _attn_sc(Q, K, V, Ep, src, dst)

    expander = jnp.kron(jnp.eye(4, dtype=jnp.float32),
                        jnp.ones((1, 32), jnp.float32))
    node_out, A, B = pl.pallas_call(
        _node_body,
        grid=(NPAD // NBLK,),
        in_specs=[
            pl.BlockSpec((NBLK, ACC_W), lambda i: (i, 0)),
            pl.BlockSpec((NBLK, ACC_W), lambda i: (i, 0)),
            pl.BlockSpec((NBLK, 128), lambda i: (i, 0)),
            pl.BlockSpec((NBLK, 128), lambda i: (i, 0)),
            _full((4, 128)),
            _full((128, 128)),
            _full((128, 128)),
            _full((1, 128)),
            _full((128, 128)),
            _full((1, 128)),
            _full((128, 256)),
            _full((1, 128)),
            _full((1, 128)),
        ],
        out_specs=[pl.BlockSpec((NBLK, 128), lambda i: (i, 0))] * 3,
        out_shape=[jax.ShapeDtypeStruct((NPAD, 128), jnp.float32)] * 3,
    )(parts[0], parts[1], nf_pad, P, expander, Wg[:128], Wg[128:],
      bg.reshape(1, 128), Wo, bo.reshape(1, 128),
      jnp.concatenate([Wu1[:128], Wu1[128:256]], axis=1), g1.reshape(1, 128),
      b1.reshape(1, 128))

    prelim = _gather_sc(A, B, src, dst)

    edge_out = pl.pallas_call(
        _edge_body,
        grid=(EPAD // EBLK,),
        in_specs=[
            pl.BlockSpec((EBLK, 128), lambda i: (i, 0)),
            pl.BlockSpec((EBLK, 16), lambda i: (i, 0)),
            _full((16, 128)),
            _full((1, 128)),
            _full((128, 16)),
            _full((1, 16)),
            _full((1, 16)),
            _full((1, 16)),
        ],
        out_specs=pl.BlockSpec((EBLK, 16), lambda i: (i, 0)),
        out_shape=jax.ShapeDtypeStruct((EPAD, 16), jnp.float32),
    )(prelim, ef_pad, Wu1[256:], bu1.reshape(1, 128), Wu2,
      bu2.reshape(1, 16), g2.reshape(1, 16), b2.reshape(1, 16))

    return node_out[:N], edge_out[:E]
